# baseline (speedup 1.0000x reference)
import functools
import math

import jax
import jax.numpy as jnp
from jax import lax
from jax.experimental import pallas as pl
from jax.experimental.pallas import tpu as pltpu
from jax.experimental.pallas import tpu_sc as plsc
from jax.scipy.special import digamma

N = 4096
D = 64
NC = 2
NS = 16
NW = NC * NS
NSC = 2048
NT = N - NSC
ROWS_PER_W = NSC // NW
RB = 8
NRB = ROWS_PER_W // RB
ROWS_PAD = ((ROWS_PER_W + 15) // 16) * 16
P = 128
NPAN = N // P
CPP = P // 16
NCH = N // 16


def _tree(vals, op):
    while len(vals) > 1:
        nxt = [op(vals[i], vals[i + 1]) for i in range(0, len(vals) - 1, 2)]
        if len(vals) % 2:
            nxt.append(vals[-1])
        vals = nxt
    return vals[0]


def _lane_max(v):
    return _tree([v[i] for i in range(16)], jnp.maximum)


def _lane_min_i32(v):
    return _tree([v[i] for i in range(16)], jnp.minimum)


def _lane_sum(v):
    return _tree([v[i] for i in range(16)], jnp.add)


def _sc_body(xp_hbm, yp_hbm, xr_hbm, yr_hbm,
             knn_hbm, nx_hbm, ny_hbm,
             xpan_v, ypan_v, myx_v, myy_v, dx_v, dy_v,
             knn_s, nx_s, ny_s, sx0, sx1, sy0, sy1):
    wid = lax.axis_index("s") * NC + lax.axis_index("c")
    row0 = wid * ROWS_PER_W
    zero = jnp.zeros((16,), jnp.float32)
    lane_iota = lax.iota(jnp.int32, 16)
    sx = (sx0, sx1)
    sy = (sy0, sy1)
    NG = D // 16

    def rb_loop(rb, carry0):
        rbase = row0 + rb * RB
        pltpu.sync_copy(xr_hbm.at[pl.ds(rbase, RB)], myx_v)
        pltpu.sync_copy(yr_hbm.at[pl.ds(rbase, RB)], myy_v)

        def pan_loop(q, carry1):
            for b in range(2):
                p = q * 2 + b
                pltpu.make_async_copy(xp_hbm.at[0], xpan_v.at[b], sx[b]).wait()
                pltpu.make_async_copy(yp_hbm.at[0], ypan_v.at[b], sy[b]).wait()

                def row_loop(r, carry2, b=b, p=p):
                    mx = [myx_v[r, pl.ds(g * 16, 16)] for g in range(NG)]
                    my = [myy_v[r, pl.ds(g * 16, 16)] for g in range(NG)]

                    UN = 8

                    def ch_loop(jc, carry3):
                        col = jc * (16 * UN)
                        ax = [zero] * UN
                        ay = [zero] * UN
                        for di in range(16):
                            for g in range(NG):
                                d = g * 16 + di
                                sxv = mx[g][di]
                                syv = my[g][di]
                                for u in range(UN):
                                    vx = xpan_v[b, d, pl.ds(col + u * 16, 16)]
                                    ax[u] = jnp.maximum(
                                        ax[u], jnp.abs(vx - sxv))
                                    vy = ypan_v[b, d, pl.ds(col + u * 16, 16)]
                                    ay[u] = jnp.maximum(
                                        ay[u], jnp.abs(vy - syv))
                        base = p * P + col
                        for u in range(UN):
                            dx_v[r, pl.ds(base + u * 16, 16)] = ax[u]
                            dy_v[r, pl.ds(base + u * 16, 16)] = ay[u]
                        return carry3

                    return lax.fori_loop(0, CPP // UN, ch_loop, carry2)

                lax.fori_loop(0, RB, row_loop, 0)

                pnext = jnp.where(p + 2 >= NPAN, p + 2 - NPAN, p + 2)
                pltpu.async_copy(xp_hbm.at[pnext], xpan_v.at[b], sx[b])
                pltpu.async_copy(yp_hbm.at[pnext], ypan_v.at[b], sy[b])
            return carry1

        lax.fori_loop(0, NPAN // 2, pan_loop, 0)

        def sel_loop(r, c):
            ka, xa, ya = c

            def t5(jq, a):
                a1, a2, a3, a4, a5 = a
                for u in range(4):
                    col = jq * 64 + u * 16
                    m = jnp.maximum(dx_v[r, pl.ds(col, 16)],
                                    dy_v[r, pl.ds(col, 16)])
                    b1 = jnp.maximum(a1, m)
                    m = jnp.minimum(a1, m)
                    b2 = jnp.maximum(a2, m)
                    m = jnp.minimum(a2, m)
                    b3 = jnp.maximum(a3, m)
                    m = jnp.minimum(a3, m)
                    b4 = jnp.maximum(a4, m)
                    m = jnp.minimum(a4, m)
                    b5 = jnp.maximum(a5, m)
                    a1, a2, a3, a4, a5 = b1, b2, b3, b4, b5
                return (a1, a2, a3, a4, a5)

            a1, a2, a3, a4, a5 = lax.fori_loop(
                0, NCH // 4, t5, (zero, zero, zero, zero, zero))
            depth = jnp.zeros((16,), jnp.int32)
            knn = jnp.float32(0.0)
            for _t in range(5):
                h = jnp.where(depth == 0, a1,
                    jnp.where(depth == 1, a2,
                    jnp.where(depth == 2, a3,
                    jnp.where(depth == 3, a4,
                    jnp.where(depth == 4, a5, jnp.float32(-1.0))))))
                knn = _lane_max(h)
                li = jnp.where(h == knn, lane_iota, jnp.int32(16))
                lstar = _lane_min_i32(li)
                depth = depth + jnp.where(lane_iota == lstar, 1, 0).astype(jnp.int32)
            thr = knn + jnp.float32(1e-15)

            def cnt(jq, cc):
                cx0, cx1, cy0, cy1 = cc
                for u in range(4):
                    col = jq * 64 + u * 16
                    vx = dx_v[r, pl.ds(col, 16)]
                    vy = dy_v[r, pl.ds(col, 16)]
                    fx = jnp.where(vx <= thr, 1.0, 0.0).astype(jnp.float32)
                    fy = jnp.where(vy <= thr, 1.0, 0.0).astype(jnp.float32)
                    if u % 2 == 0:
                        cx0 = cx0 + fx
                        cy0 = cy0 + fy
                    else:
                        cx1 = cx1 + fx
                        cy1 = cy1 + fy
                return (cx0, cx1, cy0, cy1)

            cx0, cx1, cy0, cy1 = lax.fori_loop(
                0, NCH // 4, cnt, (zero, zero, zero, zero))
            lane = (rb % 2) * 8 + r
            msk = lane_iota == lane
            ka = jnp.where(msk, knn, ka)
            xa = jnp.where(msk, _lane_sum(cx0 + cx1), xa)
            ya = jnp.where(msk, _lane_sum(cy0 + cy1), ya)
            return (ka, xa, ya)

        ka, xa, ya = lax.fori_loop(0, RB, sel_loop, carry0)

        @pl.when((rb % 2 == 1) | (rb == NRB - 1))
        def _store():
            off = (rb // 2) * 16
            knn_s[pl.ds(off, 16)] = ka
            nx_s[pl.ds(off, 16)] = xa
            ny_s[pl.ds(off, 16)] = ya

        return (ka, xa, ya)

    for b in range(2):
        pltpu.async_copy(xp_hbm.at[b], xpan_v.at[b], sx[b])
        pltpu.async_copy(yp_hbm.at[b], ypan_v.at[b], sy[b])
    lax.fori_loop(0, NRB, rb_loop, (zero, zero, zero))
    for b in range(2):
        pltpu.make_async_copy(xp_hbm.at[0], xpan_v.at[b], sx[b]).wait()
        pltpu.make_async_copy(yp_hbm.at[0], ypan_v.at[b], sy[b]).wait()
    pltpu.sync_copy(knn_s.at[pl.ds(0, ROWS_PER_W)],
                    knn_hbm.at[pl.ds(row0, ROWS_PER_W)])
    pltpu.sync_copy(nx_s.at[pl.ds(0, ROWS_PER_W)],
                    nx_hbm.at[pl.ds(row0, ROWS_PER_W)])
    pltpu.sync_copy(ny_s.at[pl.ds(0, ROWS_PER_W)],
                    ny_hbm.at[pl.ds(row0, ROWS_PER_W)])


_sc_kernel = functools.partial(
    pl.kernel,
    mesh=plsc.VectorSubcoreMesh(core_axis_name="c", subcore_axis_name="s"),
    out_type=[
        jax.ShapeDtypeStruct((NSC,), jnp.float32),
        jax.ShapeDtypeStruct((NSC,), jnp.float32),
        jax.ShapeDtypeStruct((NSC,), jnp.float32),
    ],
    scratch_types=[
        pltpu.VMEM((2, D, P), jnp.float32),
        pltpu.VMEM((2, D, P), jnp.float32),
        pltpu.VMEM((RB, D), jnp.float32),
        pltpu.VMEM((RB, D), jnp.float32),
        pltpu.VMEM((RB, N), jnp.float32),
        pltpu.VMEM((RB, N), jnp.float32),
        pltpu.VMEM((ROWS_PAD,), jnp.float32),
        pltpu.VMEM((ROWS_PAD,), jnp.float32),
        pltpu.VMEM((ROWS_PAD,), jnp.float32),
        pltpu.SemaphoreType.DMA,
        pltpu.SemaphoreType.DMA,
        pltpu.SemaphoreType.DMA,
        pltpu.SemaphoreType.DMA,
    ],
)(_sc_body)


TC_RB = 128
TC_JT = 128
TC_NJT = N // TC_JT


def _tc_body(xb_ref, yb_ref, xT_ref, yT_ref, knn_ref, nx_ref, ny_ref,
             dxb, dyb, tb):
    f32 = jnp.float32

    def jt_loop(jt, carry):
        c0 = jt * TC_JT
        xt = xT_ref[:, pl.ds(c0, TC_JT)]
        dx = jnp.zeros((TC_RB, TC_JT), f32)
        for d in range(D):
            dx = jnp.maximum(dx, jnp.abs(xb_ref[:, d:d + 1] - xt[d:d + 1, :]))
        dxb[:, pl.ds(c0, TC_JT)] = dx
        yt = yT_ref[:, pl.ds(c0, TC_JT)]
        dy = jnp.zeros((TC_RB, TC_JT), f32)
        for d in range(D):
            dy = jnp.maximum(dy, jnp.abs(yb_ref[:, d:d + 1] - yt[d:d + 1, :]))
        dyb[:, pl.ds(c0, TC_JT)] = dy
        tb[:, pl.ds(c0, TC_JT)] = jnp.maximum(dxb[:, pl.ds(c0, TC_JT)], dy)
        return carry

    lax.fori_loop(0, TC_NJT, jt_loop, 0)

    cum = jnp.zeros((TC_RB, 1), jnp.float32)
    knn = jnp.zeros((TC_RB, 1), jnp.float32)
    for _it in range(5):
        def mx_loop(jt, m):
            t = tb[:, pl.ds(jt * TC_JT, TC_JT)]
            return jnp.maximum(m, jnp.max(t, axis=1, keepdims=True))

        mval = lax.fori_loop(0, TC_NJT, mx_loop,
                             jnp.full((TC_RB, 1), -2.0, jnp.float32))

        def cm_loop(jt, c):
            t = tb[:, pl.ds(jt * TC_JT, TC_JT)]
            eq = t == mval
            tb[:, pl.ds(jt * TC_JT, TC_JT)] = jnp.where(eq, -1.0, t)
            return c + jnp.sum(eq.astype(jnp.float32), axis=1, keepdims=True)

        c = lax.fori_loop(0, TC_NJT, cm_loop,
                          jnp.zeros((TC_RB, 1), jnp.float32))
        knn = jnp.where((cum < 5.0) & (cum + c >= 5.0), mval, knn)
        cum = cum + c

    thr = knn + jnp.float32(1e-15)

    def cnt_loop(jt, cc):
        cx, cy = cc
        dx = dxb[:, pl.ds(jt * TC_JT, TC_JT)]
        dy = dyb[:, pl.ds(jt * TC_JT, TC_JT)]
        cx = cx + jnp.sum((dx <= thr).astype(jnp.float32), axis=1,
                          keepdims=True)
        cy = cy + jnp.sum((dy <= thr).astype(jnp.float32), axis=1,
                          keepdims=True)
        return (cx, cy)

    cx, cy = lax.fori_loop(0, TC_NJT, cnt_loop,
                           (jnp.zeros((TC_RB, 1), jnp.float32),
                            jnp.zeros((TC_RB, 1), jnp.float32)))
    knn_ref[...] = knn
    nx_ref[...] = cx
    ny_ref[...] = cy


def _tc_kernel(xb, yb, xT, yT):
    return pl.pallas_call(
        _tc_body,
        grid=(NT // TC_RB,),
        in_specs=[
            pl.BlockSpec((TC_RB, D), lambda i: (i, 0)),
            pl.BlockSpec((TC_RB, D), lambda i: (i, 0)),
            pl.BlockSpec((D, N), lambda i: (0, 0)),
            pl.BlockSpec((D, N), lambda i: (0, 0)),
        ],
        out_specs=[
            pl.BlockSpec((TC_RB, 1), lambda i: (i, 0)),
            pl.BlockSpec((TC_RB, 1), lambda i: (i, 0)),
            pl.BlockSpec((TC_RB, 1), lambda i: (i, 0)),
        ],
        out_shape=[
            jax.ShapeDtypeStruct((NT, 1), jnp.float32),
            jax.ShapeDtypeStruct((NT, 1), jnp.float32),
            jax.ShapeDtypeStruct((NT, 1), jnp.float32),
        ],
        scratch_shapes=[
            pltpu.VMEM((TC_RB, N), jnp.float32),
            pltpu.VMEM((TC_RB, N), jnp.float32),
            pltpu.VMEM((TC_RB, N), jnp.float32),
        ],
    )(xb, yb, xT, yT)


_LOGN = math.log(N)
_VD64 = 64.0 * math.log(2.0)
_VD128 = 128.0 * math.log(2.0)


def _fin_body(knn_ref, nx_ref, ny_ref, dig_ref, out_ref):
    lk = jnp.log(knn_ref[...])
    s1 = jnp.mean(lk)
    sx = jnp.mean(jnp.log(nx_ref[...] - 1.0))
    sy = jnp.mean(jnp.log(ny_ref[...] - 1.0))
    dig = dig_ref[0, 0]
    ans_xy = -dig + _LOGN + _VD128 + 128.0 * s1
    ans_x = _LOGN + _VD64 - sx + 64.0 * s1
    ans_y = _LOGN + _VD64 - sy + 64.0 * s1
    out_ref[...] = jnp.reshape(ans_x + ans_y - ans_xy, (1, 1))


def kernel(x_samples, y_samples, k):
    xT = x_samples.T
    yT = y_samples.T
    xp = xT.reshape(D, NPAN, P).transpose(1, 0, 2)
    yp = yT.reshape(D, NPAN, P).transpose(1, 0, 2)
    knn_sc, nx_sc, ny_sc = _sc_kernel(xp, yp, x_samples, y_samples)
    knn_tc, nx_tc, ny_tc = _tc_kernel(
        x_samples[NSC:], y_samples[NSC:], xT, yT)
    knn = jnp.concatenate([knn_sc, knn_tc[:, 0]])
    nx = jnp.concatenate([nx_sc, nx_tc[:, 0]])
    ny = jnp.concatenate([ny_sc, ny_tc[:, 0]])
    dig = digamma(jnp.asarray(k, jnp.float32)).reshape(1, 1)
    out = pl.pallas_call(
        _fin_body,
        out_shape=jax.ShapeDtypeStruct((1, 1), jnp.float32),
    )(knn.reshape(32, 128), nx.reshape(32, 128), ny.reshape(32, 128), dig)
    return out[0, 0]


# speedup vs baseline: 1.1297x; 1.1297x over previous
import functools
import math

import jax
import jax.numpy as jnp
from jax import lax
from jax.experimental import pallas as pl
from jax.experimental.pallas import tpu as pltpu
from jax.experimental.pallas import tpu_sc as plsc
from jax.scipy.special import digamma

N = 4096
D = 64
NC = 2
NS = 16
NW = NC * NS
NSC = 2304
NT = N - NSC
ROWS_PER_W = NSC // NW
RB = 8
NRB = ROWS_PER_W // RB
ROWS_PAD = ((ROWS_PER_W + 15) // 16) * 16
P = 128
NPAN = N // P
CPP = P // 16
NCH = N // 16


def _tree(vals, op):
    while len(vals) > 1:
        nxt = [op(vals[i], vals[i + 1]) for i in range(0, len(vals) - 1, 2)]
        if len(vals) % 2:
            nxt.append(vals[-1])
        vals = nxt
    return vals[0]


def _lane_max(v):
    return _tree([v[i] for i in range(16)], jnp.maximum)


def _lane_min_i32(v):
    return _tree([v[i] for i in range(16)], jnp.minimum)


def _lane_sum(v):
    return _tree([v[i] for i in range(16)], jnp.add)


def _sc_body(xp_hbm, yp_hbm, xr_hbm, yr_hbm,
             knn_hbm, nx_hbm, ny_hbm,
             xpan_v, ypan_v, myx_v, myy_v, dx_v, dy_v,
             knn_s, nx_s, ny_s, sx0, sx1, sy0, sy1):
    wid = lax.axis_index("s") * NC + lax.axis_index("c")
    row0 = wid * ROWS_PER_W
    zero = jnp.zeros((16,), jnp.float32)
    lane_iota = lax.iota(jnp.int32, 16)
    sx = (sx0, sx1)
    sy = (sy0, sy1)
    NG = D // 16

    def rb_loop(rb, carry0):
        rbase = row0 + rb * RB
        pltpu.sync_copy(xr_hbm.at[pl.ds(rbase, RB)], myx_v)
        pltpu.sync_copy(yr_hbm.at[pl.ds(rbase, RB)], myy_v)

        def pan_loop(q, carry1):
            for b in range(2):
                p = q * 2 + b
                pltpu.make_async_copy(xp_hbm.at[0], xpan_v.at[b], sx[b]).wait()
                pltpu.make_async_copy(yp_hbm.at[0], ypan_v.at[b], sy[b]).wait()

                def row_loop(r, carry2, b=b, p=p):
                    mx = [myx_v[r, pl.ds(g * 16, 16)] for g in range(NG)]
                    my = [myy_v[r, pl.ds(g * 16, 16)] for g in range(NG)]

                    UN = 8

                    def ch_loop(jc, carry3):
                        col = jc * (16 * UN)
                        ax = [zero] * UN
                        ay = [zero] * UN
                        for di in range(16):
                            for g in range(NG):
                                d = g * 16 + di
                                sxv = mx[g][di]
                                syv = my[g][di]
                                for u in range(UN):
                                    vx = xpan_v[b, d, pl.ds(col + u * 16, 16)]
                                    ax[u] = jnp.maximum(
                                        ax[u], jnp.abs(vx - sxv))
                                    vy = ypan_v[b, d, pl.ds(col + u * 16, 16)]
                                    ay[u] = jnp.maximum(
                                        ay[u], jnp.abs(vy - syv))
                        base = p * P + col
                        for u in range(UN):
                            dx_v[r, pl.ds(base + u * 16, 16)] = ax[u]
                            dy_v[r, pl.ds(base + u * 16, 16)] = ay[u]
                        return carry3

                    return lax.fori_loop(0, CPP // UN, ch_loop, carry2)

                lax.fori_loop(0, RB, row_loop, 0)

                pnext = jnp.where(p + 2 >= NPAN, p + 2 - NPAN, p + 2)
                pltpu.async_copy(xp_hbm.at[pnext], xpan_v.at[b], sx[b])
                pltpu.async_copy(yp_hbm.at[pnext], ypan_v.at[b], sy[b])
            return carry1

        lax.fori_loop(0, NPAN // 2, pan_loop, 0)

        def sel_loop(r, c):
            ka, xa, ya = c

            def t5(jq, a):
                a1, a2, a3, a4, a5 = a
                for u in range(4):
                    col = jq * 64 + u * 16
                    m = jnp.maximum(dx_v[r, pl.ds(col, 16)],
                                    dy_v[r, pl.ds(col, 16)])
                    b1 = jnp.maximum(a1, m)
                    m = jnp.minimum(a1, m)
                    b2 = jnp.maximum(a2, m)
                    m = jnp.minimum(a2, m)
                    b3 = jnp.maximum(a3, m)
                    m = jnp.minimum(a3, m)
                    b4 = jnp.maximum(a4, m)
                    m = jnp.minimum(a4, m)
                    b5 = jnp.maximum(a5, m)
                    a1, a2, a3, a4, a5 = b1, b2, b3, b4, b5
                return (a1, a2, a3, a4, a5)

            a1, a2, a3, a4, a5 = lax.fori_loop(
                0, NCH // 4, t5, (zero, zero, zero, zero, zero))
            depth = jnp.zeros((16,), jnp.int32)
            knn = jnp.float32(0.0)
            for _t in range(5):
                h = jnp.where(depth == 0, a1,
                    jnp.where(depth == 1, a2,
                    jnp.where(depth == 2, a3,
                    jnp.where(depth == 3, a4,
                    jnp.where(depth == 4, a5, jnp.float32(-1.0))))))
                knn = _lane_max(h)
                li = jnp.where(h == knn, lane_iota, jnp.int32(16))
                lstar = _lane_min_i32(li)
                depth = depth + jnp.where(lane_iota == lstar, 1, 0).astype(jnp.int32)
            thr = knn + jnp.float32(1e-15)

            def cnt(jq, cc):
                cx0, cx1, cy0, cy1 = cc
                for u in range(4):
                    col = jq * 64 + u * 16
                    vx = dx_v[r, pl.ds(col, 16)]
                    vy = dy_v[r, pl.ds(col, 16)]
                    fx = jnp.where(vx <= thr, 1.0, 0.0).astype(jnp.float32)
                    fy = jnp.where(vy <= thr, 1.0, 0.0).astype(jnp.float32)
                    if u % 2 == 0:
                        cx0 = cx0 + fx
                        cy0 = cy0 + fy
                    else:
                        cx1 = cx1 + fx
                        cy1 = cy1 + fy
                return (cx0, cx1, cy0, cy1)

            cx0, cx1, cy0, cy1 = lax.fori_loop(
                0, NCH // 4, cnt, (zero, zero, zero, zero))
            lane = (rb % 2) * 8 + r
            msk = lane_iota == lane
            ka = jnp.where(msk, knn, ka)
            xa = jnp.where(msk, _lane_sum(cx0 + cx1), xa)
            ya = jnp.where(msk, _lane_sum(cy0 + cy1), ya)
            return (ka, xa, ya)

        ka, xa, ya = lax.fori_loop(0, RB, sel_loop, carry0)

        @pl.when((rb % 2 == 1) | (rb == NRB - 1))
        def _store():
            off = (rb // 2) * 16
            knn_s[pl.ds(off, 16)] = ka
            nx_s[pl.ds(off, 16)] = xa
            ny_s[pl.ds(off, 16)] = ya

        return (ka, xa, ya)

    for b in range(2):
        pltpu.async_copy(xp_hbm.at[b], xpan_v.at[b], sx[b])
        pltpu.async_copy(yp_hbm.at[b], ypan_v.at[b], sy[b])
    lax.fori_loop(0, NRB, rb_loop, (zero, zero, zero))
    for b in range(2):
        pltpu.make_async_copy(xp_hbm.at[0], xpan_v.at[b], sx[b]).wait()
        pltpu.make_async_copy(yp_hbm.at[0], ypan_v.at[b], sy[b]).wait()
    pltpu.sync_copy(knn_s.at[pl.ds(0, ROWS_PER_W)],
                    knn_hbm.at[pl.ds(row0, ROWS_PER_W)])
    pltpu.sync_copy(nx_s.at[pl.ds(0, ROWS_PER_W)],
                    nx_hbm.at[pl.ds(row0, ROWS_PER_W)])
    pltpu.sync_copy(ny_s.at[pl.ds(0, ROWS_PER_W)],
                    ny_hbm.at[pl.ds(row0, ROWS_PER_W)])


_sc_kernel = functools.partial(
    pl.kernel,
    mesh=plsc.VectorSubcoreMesh(core_axis_name="c", subcore_axis_name="s"),
    out_type=[
        jax.ShapeDtypeStruct((NSC,), jnp.float32),
        jax.ShapeDtypeStruct((NSC,), jnp.float32),
        jax.ShapeDtypeStruct((NSC,), jnp.float32),
    ],
    scratch_types=[
        pltpu.VMEM((2, D, P), jnp.float32),
        pltpu.VMEM((2, D, P), jnp.float32),
        pltpu.VMEM((RB, D), jnp.float32),
        pltpu.VMEM((RB, D), jnp.float32),
        pltpu.VMEM((RB, N), jnp.float32),
        pltpu.VMEM((RB, N), jnp.float32),
        pltpu.VMEM((ROWS_PAD,), jnp.float32),
        pltpu.VMEM((ROWS_PAD,), jnp.float32),
        pltpu.VMEM((ROWS_PAD,), jnp.float32),
        pltpu.SemaphoreType.DMA,
        pltpu.SemaphoreType.DMA,
        pltpu.SemaphoreType.DMA,
        pltpu.SemaphoreType.DMA,
    ],
)(_sc_body)


TC_RB = 128
TC_JT = 128
TC_NJT = N // TC_JT


def _tc_body(xb_ref, yb_ref, xT_ref, yT_ref, knn_ref, nx_ref, ny_ref,
             dxb, dyb, tb, bxb, byb):
    f32 = jnp.float32

    for d in range(D):
        bxb[d] = jnp.broadcast_to(xb_ref[:, d:d + 1], (TC_RB, TC_JT))
        byb[d] = jnp.broadcast_to(yb_ref[:, d:d + 1], (TC_RB, TC_JT))

    def jt_loop(jt, carry):
        c0 = jt * TC_JT
        xt = xT_ref[:, pl.ds(c0, TC_JT)]
        dx = jnp.zeros((TC_RB, TC_JT), f32)
        for d in range(D):
            dx = jnp.maximum(dx, jnp.abs(bxb[d] - xt[d:d + 1, :]))
        dxb[:, pl.ds(c0, TC_JT)] = dx
        yt = yT_ref[:, pl.ds(c0, TC_JT)]
        dy = jnp.zeros((TC_RB, TC_JT), f32)
        for d in range(D):
            dy = jnp.maximum(dy, jnp.abs(byb[d] - yt[d:d + 1, :]))
        dyb[:, pl.ds(c0, TC_JT)] = dy
        tb[:, pl.ds(c0, TC_JT)] = jnp.maximum(dxb[:, pl.ds(c0, TC_JT)], dy)
        return carry

    lax.fori_loop(0, TC_NJT, jt_loop, 0)

    cum = jnp.zeros((TC_RB, 1), jnp.float32)
    knn = jnp.zeros((TC_RB, 1), jnp.float32)
    for _it in range(5):
        def mx_loop(jt, m):
            t = tb[:, pl.ds(jt * TC_JT, TC_JT)]
            return jnp.maximum(m, jnp.max(t, axis=1, keepdims=True))

        mval = lax.fori_loop(0, TC_NJT, mx_loop,
                             jnp.full((TC_RB, 1), -2.0, jnp.float32))

        def cm_loop(jt, c):
            t = tb[:, pl.ds(jt * TC_JT, TC_JT)]
            eq = t == mval
            tb[:, pl.ds(jt * TC_JT, TC_JT)] = jnp.where(eq, -1.0, t)
            return c + jnp.sum(eq.astype(jnp.float32), axis=1, keepdims=True)

        c = lax.fori_loop(0, TC_NJT, cm_loop,
                          jnp.zeros((TC_RB, 1), jnp.float32))
        knn = jnp.where((cum < 5.0) & (cum + c >= 5.0), mval, knn)
        cum = cum + c

    thr = knn + jnp.float32(1e-15)

    def cnt_loop(jt, cc):
        cx, cy = cc
        dx = dxb[:, pl.ds(jt * TC_JT, TC_JT)]
        dy = dyb[:, pl.ds(jt * TC_JT, TC_JT)]
        cx = cx + jnp.sum((dx <= thr).astype(jnp.float32), axis=1,
                          keepdims=True)
        cy = cy + jnp.sum((dy <= thr).astype(jnp.float32), axis=1,
                          keepdims=True)
        return (cx, cy)

    cx, cy = lax.fori_loop(0, TC_NJT, cnt_loop,
                           (jnp.zeros((TC_RB, 1), jnp.float32),
                            jnp.zeros((TC_RB, 1), jnp.float32)))
    knn_ref[...] = knn
    nx_ref[...] = cx
    ny_ref[...] = cy


def _tc_kernel(xb, yb, xT, yT):
    return pl.pallas_call(
        _tc_body,
        grid=(NT // TC_RB,),
        in_specs=[
            pl.BlockSpec((TC_RB, D), lambda i: (i, 0)),
            pl.BlockSpec((TC_RB, D), lambda i: (i, 0)),
            pl.BlockSpec((D, N), lambda i: (0, 0)),
            pl.BlockSpec((D, N), lambda i: (0, 0)),
        ],
        out_specs=[
            pl.BlockSpec((TC_RB, 1), lambda i: (i, 0)),
            pl.BlockSpec((TC_RB, 1), lambda i: (i, 0)),
            pl.BlockSpec((TC_RB, 1), lambda i: (i, 0)),
        ],
        out_shape=[
            jax.ShapeDtypeStruct((NT, 1), jnp.float32),
            jax.ShapeDtypeStruct((NT, 1), jnp.float32),
            jax.ShapeDtypeStruct((NT, 1), jnp.float32),
        ],
        scratch_shapes=[
            pltpu.VMEM((TC_RB, N), jnp.float32),
            pltpu.VMEM((TC_RB, N), jnp.float32),
            pltpu.VMEM((TC_RB, N), jnp.float32),
            pltpu.VMEM((D, TC_RB, TC_JT), jnp.float32),
            pltpu.VMEM((D, TC_RB, TC_JT), jnp.float32),
        ],
    )(xb, yb, xT, yT)


_LOGN = math.log(N)
_VD64 = 64.0 * math.log(2.0)
_VD128 = 128.0 * math.log(2.0)


def _fin_body(knn_ref, nx_ref, ny_ref, dig_ref, out_ref):
    lk = jnp.log(knn_ref[...])
    s1 = jnp.mean(lk)
    sx = jnp.mean(jnp.log(nx_ref[...] - 1.0))
    sy = jnp.mean(jnp.log(ny_ref[...] - 1.0))
    dig = dig_ref[0, 0]
    ans_xy = -dig + _LOGN + _VD128 + 128.0 * s1
    ans_x = _LOGN + _VD64 - sx + 64.0 * s1
    ans_y = _LOGN + _VD64 - sy + 64.0 * s1
    out_ref[...] = jnp.reshape(ans_x + ans_y - ans_xy, (1, 1))


def kernel(x_samples, y_samples, k):
    xT = x_samples.T
    yT = y_samples.T
    xp = xT.reshape(D, NPAN, P).transpose(1, 0, 2)
    yp = yT.reshape(D, NPAN, P).transpose(1, 0, 2)
    knn_sc, nx_sc, ny_sc = _sc_kernel(xp, yp, x_samples, y_samples)
    knn_tc, nx_tc, ny_tc = _tc_kernel(
        x_samples[NSC:], y_samples[NSC:], xT, yT)
    knn = jnp.concatenate([knn_sc, knn_tc[:, 0]])
    nx = jnp.concatenate([nx_sc, nx_tc[:, 0]])
    ny = jnp.concatenate([ny_sc, ny_tc[:, 0]])
    dig = digamma(jnp.asarray(k, jnp.float32)).reshape(1, 1)
    out = pl.pallas_call(
        _fin_body,
        out_shape=jax.ShapeDtypeStruct((1, 1), jnp.float32),
    )(knn.reshape(32, 128), nx.reshape(32, 128), ny.reshape(32, 128), dig)
    return out[0, 0]


# speedup vs baseline: 1.2645x; 1.1193x over previous
import functools
import math

import jax
import jax.numpy as jnp
from jax import lax
from jax.experimental import pallas as pl
from jax.experimental.pallas import tpu as pltpu
from jax.experimental.pallas import tpu_sc as plsc
from jax.scipy.special import digamma

N = 4096
D = 64
NC = 2
NS = 16
NW = NC * NS
NSC = 2048
NT = N - NSC
ROWS_PER_W = NSC // NW
RB = 8
NRB = ROWS_PER_W // RB
ROWS_PAD = ((ROWS_PER_W + 15) // 16) * 16
P = 128
NPAN = N // P
CPP = P // 16
NCH = N // 16


def _tree(vals, op):
    while len(vals) > 1:
        nxt = [op(vals[i], vals[i + 1]) for i in range(0, len(vals) - 1, 2)]
        if len(vals) % 2:
            nxt.append(vals[-1])
        vals = nxt
    return vals[0]


def _lane_max(v):
    return _tree([v[i] for i in range(16)], jnp.maximum)


def _lane_min_i32(v):
    return _tree([v[i] for i in range(16)], jnp.minimum)


def _lane_sum(v):
    return _tree([v[i] for i in range(16)], jnp.add)


def _sc_body(xp_hbm, yp_hbm, xr_hbm, yr_hbm,
             knn_hbm, nx_hbm, ny_hbm,
             xpan_v, ypan_v, myx_v, myy_v, dx_v, dy_v,
             knn_s, nx_s, ny_s, sx0, sx1, sy0, sy1):
    wid = lax.axis_index("s") * NC + lax.axis_index("c")
    row0 = wid * ROWS_PER_W
    zero = jnp.zeros((16,), jnp.float32)
    lane_iota = lax.iota(jnp.int32, 16)
    sx = (sx0, sx1)
    sy = (sy0, sy1)
    NG = D // 16

    def rb_loop(rb, carry0):
        rbase = row0 + rb * RB
        pltpu.sync_copy(xr_hbm.at[pl.ds(rbase, RB)], myx_v)
        pltpu.sync_copy(yr_hbm.at[pl.ds(rbase, RB)], myy_v)

        def pan_loop(q, carry1):
            for b in range(2):
                p = q * 2 + b
                pltpu.make_async_copy(xp_hbm.at[0], xpan_v.at[b], sx[b]).wait()
                pltpu.make_async_copy(yp_hbm.at[0], ypan_v.at[b], sy[b]).wait()

                def row_loop(r, carry2, b=b, p=p):
                    mx = [myx_v[r, pl.ds(g * 16, 16)] for g in range(NG)]
                    my = [myy_v[r, pl.ds(g * 16, 16)] for g in range(NG)]

                    UN = 8

                    def ch_loop(jc, carry3):
                        col = jc * (16 * UN)
                        ax = [zero] * UN
                        ay = [zero] * UN
                        for di in range(16):
                            for g in range(NG):
                                d = g * 16 + di
                                sxv = mx[g][di]
                                syv = my[g][di]
                                for u in range(UN):
                                    vx = xpan_v[b, d, pl.ds(col + u * 16, 16)]
                                    ax[u] = jnp.maximum(
                                        ax[u], jnp.abs(vx - sxv))
                                    vy = ypan_v[b, d, pl.ds(col + u * 16, 16)]
                                    ay[u] = jnp.maximum(
                                        ay[u], jnp.abs(vy - syv))
                        base = p * P + col
                        for u in range(UN):
                            dx_v[r, pl.ds(base + u * 16, 16)] = ax[u]
                            dy_v[r, pl.ds(base + u * 16, 16)] = ay[u]
                        return carry3

                    return lax.fori_loop(0, CPP // UN, ch_loop, carry2)

                lax.fori_loop(0, RB, row_loop, 0)

                pnext = jnp.where(p + 2 >= NPAN, p + 2 - NPAN, p + 2)
                pltpu.async_copy(xp_hbm.at[pnext], xpan_v.at[b], sx[b])
                pltpu.async_copy(yp_hbm.at[pnext], ypan_v.at[b], sy[b])
            return carry1

        lax.fori_loop(0, NPAN // 2, pan_loop, 0)

        def sel_loop(r, c):
            ka, xa, ya = c

            def t5(jq, a):
                a1, a2, a3, a4, a5 = a
                for u in range(4):
                    col = jq * 64 + u * 16
                    m = jnp.maximum(dx_v[r, pl.ds(col, 16)],
                                    dy_v[r, pl.ds(col, 16)])
                    b1 = jnp.maximum(a1, m)
                    m = jnp.minimum(a1, m)
                    b2 = jnp.maximum(a2, m)
                    m = jnp.minimum(a2, m)
                    b3 = jnp.maximum(a3, m)
                    m = jnp.minimum(a3, m)
                    b4 = jnp.maximum(a4, m)
                    m = jnp.minimum(a4, m)
                    b5 = jnp.maximum(a5, m)
                    a1, a2, a3, a4, a5 = b1, b2, b3, b4, b5
                return (a1, a2, a3, a4, a5)

            a1, a2, a3, a4, a5 = lax.fori_loop(
                0, NCH // 4, t5, (zero, zero, zero, zero, zero))
            depth = jnp.zeros((16,), jnp.int32)
            knn = jnp.float32(0.0)
            for _t in range(5):
                h = jnp.where(depth == 0, a1,
                    jnp.where(depth == 1, a2,
                    jnp.where(depth == 2, a3,
                    jnp.where(depth == 3, a4,
                    jnp.where(depth == 4, a5, jnp.float32(-1.0))))))
                knn = _lane_max(h)
                li = jnp.where(h == knn, lane_iota, jnp.int32(16))
                lstar = _lane_min_i32(li)
                depth = depth + jnp.where(lane_iota == lstar, 1, 0).astype(jnp.int32)
            thr = knn + jnp.float32(1e-15)

            def cnt(jq, cc):
                cx0, cx1, cy0, cy1 = cc
                for u in range(4):
                    col = jq * 64 + u * 16
                    vx = dx_v[r, pl.ds(col, 16)]
                    vy = dy_v[r, pl.ds(col, 16)]
                    fx = jnp.where(vx <= thr, 1.0, 0.0).astype(jnp.float32)
                    fy = jnp.where(vy <= thr, 1.0, 0.0).astype(jnp.float32)
                    if u % 2 == 0:
                        cx0 = cx0 + fx
                        cy0 = cy0 + fy
                    else:
                        cx1 = cx1 + fx
                        cy1 = cy1 + fy
                return (cx0, cx1, cy0, cy1)

            cx0, cx1, cy0, cy1 = lax.fori_loop(
                0, NCH // 4, cnt, (zero, zero, zero, zero))
            lane = (rb % 2) * 8 + r
            msk = lane_iota == lane
            ka = jnp.where(msk, knn, ka)
            xa = jnp.where(msk, _lane_sum(cx0 + cx1), xa)
            ya = jnp.where(msk, _lane_sum(cy0 + cy1), ya)
            return (ka, xa, ya)

        ka, xa, ya = lax.fori_loop(0, RB, sel_loop, carry0)

        @pl.when((rb % 2 == 1) | (rb == NRB - 1))
        def _store():
            off = (rb // 2) * 16
            knn_s[pl.ds(off, 16)] = ka
            nx_s[pl.ds(off, 16)] = xa
            ny_s[pl.ds(off, 16)] = ya

        return (ka, xa, ya)

    for b in range(2):
        pltpu.async_copy(xp_hbm.at[b], xpan_v.at[b], sx[b])
        pltpu.async_copy(yp_hbm.at[b], ypan_v.at[b], sy[b])
    lax.fori_loop(0, NRB, rb_loop, (zero, zero, zero))
    for b in range(2):
        pltpu.make_async_copy(xp_hbm.at[0], xpan_v.at[b], sx[b]).wait()
        pltpu.make_async_copy(yp_hbm.at[0], ypan_v.at[b], sy[b]).wait()
    pltpu.sync_copy(knn_s.at[pl.ds(0, ROWS_PER_W)],
                    knn_hbm.at[pl.ds(row0, ROWS_PER_W)])
    pltpu.sync_copy(nx_s.at[pl.ds(0, ROWS_PER_W)],
                    nx_hbm.at[pl.ds(row0, ROWS_PER_W)])
    pltpu.sync_copy(ny_s.at[pl.ds(0, ROWS_PER_W)],
                    ny_hbm.at[pl.ds(row0, ROWS_PER_W)])


_sc_kernel = functools.partial(
    pl.kernel,
    mesh=plsc.VectorSubcoreMesh(core_axis_name="c", subcore_axis_name="s"),
    out_type=[
        jax.ShapeDtypeStruct((NSC,), jnp.float32),
        jax.ShapeDtypeStruct((NSC,), jnp.float32),
        jax.ShapeDtypeStruct((NSC,), jnp.float32),
    ],
    scratch_types=[
        pltpu.VMEM((2, D, P), jnp.float32),
        pltpu.VMEM((2, D, P), jnp.float32),
        pltpu.VMEM((RB, D), jnp.float32),
        pltpu.VMEM((RB, D), jnp.float32),
        pltpu.VMEM((RB, N), jnp.float32),
        pltpu.VMEM((RB, N), jnp.float32),
        pltpu.VMEM((ROWS_PAD,), jnp.float32),
        pltpu.VMEM((ROWS_PAD,), jnp.float32),
        pltpu.VMEM((ROWS_PAD,), jnp.float32),
        pltpu.SemaphoreType.DMA,
        pltpu.SemaphoreType.DMA,
        pltpu.SemaphoreType.DMA,
        pltpu.SemaphoreType.DMA,
    ],
)(_sc_body)


TC_RB = 128
TC_JT = 128
TC_NJT = N // TC_JT


def _tc_body(xb_ref, yb_ref, xT_ref, yT_ref, knn_ref, nx_ref, ny_ref,
             dxb, dyb, tb, bxb, byb):
    f32 = jnp.float32

    for d in range(D):
        bxb[d] = jnp.broadcast_to(xb_ref[:, d:d + 1], (TC_RB, TC_JT))
        byb[d] = jnp.broadcast_to(yb_ref[:, d:d + 1], (TC_RB, TC_JT))

    def jt_loop(jt, carry):
        c0 = jt * TC_JT
        xt = xT_ref[:, pl.ds(c0, TC_JT)]
        dx = jnp.zeros((TC_RB, TC_JT), f32)
        for d in range(D):
            dx = jnp.maximum(dx, jnp.abs(bxb[d] - xt[d:d + 1, :]))
        dxb[:, pl.ds(c0, TC_JT)] = dx
        yt = yT_ref[:, pl.ds(c0, TC_JT)]
        dy = jnp.zeros((TC_RB, TC_JT), f32)
        for d in range(D):
            dy = jnp.maximum(dy, jnp.abs(byb[d] - yt[d:d + 1, :]))
        dyb[:, pl.ds(c0, TC_JT)] = dy
        tb[:, pl.ds(c0, TC_JT)] = jnp.maximum(dxb[:, pl.ds(c0, TC_JT)], dy)
        return carry

    lax.fori_loop(0, TC_NJT, jt_loop, 0)

    cum = jnp.zeros((TC_RB, 1), jnp.float32)
    knn = jnp.zeros((TC_RB, 1), jnp.float32)
    for _it in range(5):
        def mx_loop(jt, m):
            t = tb[:, pl.ds(jt * TC_JT, TC_JT)]
            return jnp.maximum(m, jnp.max(t, axis=1, keepdims=True))

        mval = lax.fori_loop(0, TC_NJT, mx_loop,
                             jnp.full((TC_RB, 1), -2.0, jnp.float32))

        def cm_loop(jt, c):
            t = tb[:, pl.ds(jt * TC_JT, TC_JT)]
            eq = t == mval
            tb[:, pl.ds(jt * TC_JT, TC_JT)] = jnp.where(eq, -1.0, t)
            return c + jnp.sum(eq.astype(jnp.float32), axis=1, keepdims=True)

        c = lax.fori_loop(0, TC_NJT, cm_loop,
                          jnp.zeros((TC_RB, 1), jnp.float32))
        knn = jnp.where((cum < 5.0) & (cum + c >= 5.0), mval, knn)
        cum = cum + c

    thr = knn + jnp.float32(1e-15)

    def cnt_loop(jt, cc):
        cx, cy = cc
        dx = dxb[:, pl.ds(jt * TC_JT, TC_JT)]
        dy = dyb[:, pl.ds(jt * TC_JT, TC_JT)]
        cx = cx + jnp.sum((dx <= thr).astype(jnp.float32), axis=1,
                          keepdims=True)
        cy = cy + jnp.sum((dy <= thr).astype(jnp.float32), axis=1,
                          keepdims=True)
        return (cx, cy)

    cx, cy = lax.fori_loop(0, TC_NJT, cnt_loop,
                           (jnp.zeros((TC_RB, 1), jnp.float32),
                            jnp.zeros((TC_RB, 1), jnp.float32)))
    knn_ref[...] = knn
    nx_ref[...] = cx
    ny_ref[...] = cy


def _tc_kernel(xb, yb, xT, yT):
    return pl.pallas_call(
        _tc_body,
        grid=(NT // TC_RB,),
        in_specs=[
            pl.BlockSpec((TC_RB, D), lambda i: (i, 0)),
            pl.BlockSpec((TC_RB, D), lambda i: (i, 0)),
            pl.BlockSpec((D, N), lambda i: (0, 0)),
            pl.BlockSpec((D, N), lambda i: (0, 0)),
        ],
        out_specs=[
            pl.BlockSpec((TC_RB, 1), lambda i: (i, 0)),
            pl.BlockSpec((TC_RB, 1), lambda i: (i, 0)),
            pl.BlockSpec((TC_RB, 1), lambda i: (i, 0)),
        ],
        out_shape=[
            jax.ShapeDtypeStruct((NT, 1), jnp.float32),
            jax.ShapeDtypeStruct((NT, 1), jnp.float32),
            jax.ShapeDtypeStruct((NT, 1), jnp.float32),
        ],
        scratch_shapes=[
            pltpu.VMEM((TC_RB, N), jnp.float32),
            pltpu.VMEM((TC_RB, N), jnp.float32),
            pltpu.VMEM((TC_RB, N), jnp.float32),
            pltpu.VMEM((D, TC_RB, TC_JT), jnp.float32),
            pltpu.VMEM((D, TC_RB, TC_JT), jnp.float32),
        ],
    )(xb, yb, xT, yT)


_LOGN = math.log(N)
_VD64 = 64.0 * math.log(2.0)
_VD128 = 128.0 * math.log(2.0)


def _fin_body(knn_ref, nx_ref, ny_ref, dig_ref, out_ref):
    lk = jnp.log(knn_ref[...])
    s1 = jnp.mean(lk)
    sx = jnp.mean(jnp.log(nx_ref[...] - 1.0))
    sy = jnp.mean(jnp.log(ny_ref[...] - 1.0))
    dig = dig_ref[0, 0]
    ans_xy = -dig + _LOGN + _VD128 + 128.0 * s1
    ans_x = _LOGN + _VD64 - sx + 64.0 * s1
    ans_y = _LOGN + _VD64 - sy + 64.0 * s1
    out_ref[...] = jnp.reshape(ans_x + ans_y - ans_xy, (1, 1))


def kernel(x_samples, y_samples, k):
    xT = x_samples.T
    yT = y_samples.T
    xp = xT.reshape(D, NPAN, P).transpose(1, 0, 2)
    yp = yT.reshape(D, NPAN, P).transpose(1, 0, 2)
    knn_sc, nx_sc, ny_sc = _sc_kernel(xp, yp, x_samples, y_samples)
    knn_tc, nx_tc, ny_tc = _tc_kernel(
        x_samples[NSC:], y_samples[NSC:], xT, yT)
    knn = jnp.concatenate([knn_sc, knn_tc[:, 0]])
    nx = jnp.concatenate([nx_sc, nx_tc[:, 0]])
    ny = jnp.concatenate([ny_sc, ny_tc[:, 0]])
    dig = digamma(jnp.asarray(k, jnp.float32)).reshape(1, 1)
    out = pl.pallas_call(
        _fin_body,
        out_shape=jax.ShapeDtypeStruct((1, 1), jnp.float32),
    )(knn.reshape(32, 128), nx.reshape(32, 128), ny.reshape(32, 128), dig)
    return out[0, 0]
